# baseline (speedup 1.0000x reference)
import jax
import jax.numpy as jnp
from jax.experimental import pallas as pl
from jax.experimental.pallas import tpu as pltpu

_LANE = 128
_SH = 56


def _conv3x3(src_f, rhs, m, w):
    c = _LANE
    p = jnp.dot(src_f[0:m + 2 * w], rhs, preferred_element_type=jnp.float32)
    return (p[0:m, 0:c] + p[w:m + w, c:2 * c] +
            p[2 * w:m + 2 * w, 2 * c:3 * c])


def _kx_expand(src, w, c):
    rows = src.shape[0]
    zc = jnp.zeros((rows, 1, c), src.dtype)
    left = jnp.concatenate([zc, src[:, 0:w - 1, :]], axis=1)
    right = jnp.concatenate([src[:, 1:w, :], zc], axis=1)
    return jnp.concatenate([left, src, right], axis=-1)


def _subtile(xs, ident, rhs1, b1, rhs2, b2, r0, sh, w, h):
    c = _LANE

    xw3 = _kx_expand(xs, w, c).reshape((sh + 4) * w, 3 * c)
    m1 = (sh + 2) * w
    out1 = jnp.maximum(_conv3x3(xw3, rhs1, m1, w) + b1, 0.0)
    out1 = out1.reshape(sh + 2, w, c).astype(jnp.bfloat16)

    g = jax.lax.broadcasted_iota(jnp.int32, (sh + 2, 1, 1), 0) + (r0 - 1)
    out1 = jnp.where((g >= 0) & (g < h), out1, jnp.bfloat16(0))

    mw3 = _kx_expand(out1, w, c).reshape((sh + 2) * w, 3 * c)
    m2 = sh * w
    acc2 = _conv3x3(mw3, rhs2, m2, w)

    return jnp.maximum(acc2 + b2 + ident, 0.0)


def _block_body(x_ref, r1_ref, b1_ref, r2_ref, b2_ref, o_ref, *, sh, w, h):
    c = _LANE
    hw = h * w
    hwp = ((hw + c - 1) // c) * c

    xf = x_ref[...]
    xfp = jnp.concatenate([xf, jnp.zeros((c, hwp - hw), jnp.float32)], axis=1)
    xt = jnp.transpose(xfp)
    xs_all = xt[0:hw].astype(jnp.bfloat16).reshape(h, w, c)
    zr = jnp.zeros((2, w, c), jnp.bfloat16)
    xs_pad = jnp.concatenate([zr, xs_all, zr], axis=0)

    ws = (r1_ref[...], b1_ref[0], r2_ref[...], b2_ref[0])
    results = []
    for t in range(h // sh):
        xs = xs_pad[t * sh:t * sh + sh + 4]
        ident = xt[t * sh * w:(t + 1) * sh * w]
        results.append(_subtile(xs, ident, *ws, t * sh, sh, w, h))
    res = results[0] if len(results) == 1 else jnp.concatenate(results, axis=0)

    resp = jnp.concatenate([res, jnp.zeros((hwp - hw, c), jnp.float32)],
                           axis=0)
    o_ref[...] = jnp.transpose(resp)[:, 0:hw]


def _basic_block(x_nchw, w1, g1, be1, m1, v1, w2, g2, be2, m2, v2, eps=1e-5):
    B, C, H, W = x_nchw.shape
    assert C == _LANE and w1.shape[0] == C
    sh = _SH if H % _SH == 0 else H

    s1 = g1 / jnp.sqrt(v1 + eps)
    s2 = g2 / jnp.sqrt(v2 + eps)
    b1 = (be1 - m1 * s1).reshape(1, C).astype(jnp.float32)
    b2 = (be2 - m2 * s2).reshape(1, C).astype(jnp.float32)

    def prep_w(wt, s):
        whwio = jnp.transpose(wt, (2, 3, 1, 0)) * s[None, None, None, :]
        per_ky = whwio.reshape(3, 3 * C, C).astype(jnp.bfloat16)
        return jnp.concatenate([per_ky[0], per_ky[1], per_ky[2]], axis=1)

    w1f = prep_w(w1, s1)
    w2f = prep_w(w2, s2)

    xflat = jnp.reshape(x_nchw, (B, C, H * W))

    def body(*refs):
        return _block_body(*refs, sh=sh, w=W, h=H)

    out = pl.pallas_call(
        body,
        out_shape=jax.ShapeDtypeStruct((B, C, H * W), jnp.float32),
        grid_spec=pltpu.PrefetchScalarGridSpec(
            num_scalar_prefetch=0,
            grid=(B,),
            in_specs=[
                pl.BlockSpec((None, C, H * W), lambda b: (b, 0, 0)),
                pl.BlockSpec((3 * C, 3 * C), lambda b: (0, 0)),
                pl.BlockSpec((1, C), lambda b: (0, 0)),
                pl.BlockSpec((3 * C, 3 * C), lambda b: (0, 0)),
                pl.BlockSpec((1, C), lambda b: (0, 0)),
            ],
            out_specs=pl.BlockSpec((None, C, H * W), lambda b: (b, 0, 0)),
        ),
        compiler_params=pltpu.CompilerParams(
            dimension_semantics=("parallel",)),
    )(xflat, w1f, b1, w2f, b2)

    return jnp.reshape(out, (B, C, H, W))


def kernel(x, w1, g1, be1, m1, v1, w2, g2, be2, m2, v2):
    return _basic_block(x, w1, g1, be1, m1, v1, w2, g2, be2, m2, v2)


# speedup vs baseline: 1.2494x; 1.2494x over previous
import jax
import jax.numpy as jnp
from jax.experimental import pallas as pl
from jax.experimental.pallas import tpu as pltpu

_LANE = 128
_SH = 28


def _conv3x3(src_f, rhs, m, w):
    c = _LANE
    p = jnp.dot(src_f[0:m + 2 * w], rhs, preferred_element_type=jnp.float32)
    return (p[0:m, 0:c] + p[w:m + w, c:2 * c] +
            p[2 * w:m + 2 * w, 2 * c:3 * c])


def _kx_expand(src, w, c):
    rows = src.shape[0]
    zc = jnp.zeros((rows, 1, c), src.dtype)
    left = jnp.concatenate([zc, src[:, 0:w - 1, :]], axis=1)
    right = jnp.concatenate([src[:, 1:w, :], zc], axis=1)
    return jnp.concatenate([left, src, right], axis=-1)


def _subtile(xs, ident, rhs1, b1, rhs2, b2, r0, sh, w, h):
    c = _LANE

    xw3 = _kx_expand(xs, w, c).reshape((sh + 4) * w, 3 * c)
    m1 = (sh + 2) * w
    out1 = jnp.maximum(_conv3x3(xw3, rhs1, m1, w) + b1, 0.0)
    out1 = out1.reshape(sh + 2, w, c).astype(jnp.bfloat16)

    g = jax.lax.broadcasted_iota(jnp.int32, (sh + 2, 1, 1), 0) + (r0 - 1)
    out1 = jnp.where((g >= 0) & (g < h), out1, jnp.bfloat16(0))

    mw3 = _kx_expand(out1, w, c).reshape((sh + 2) * w, 3 * c)
    m2 = sh * w
    acc2 = _conv3x3(mw3, rhs2, m2, w)

    return jnp.maximum(acc2 + b2 + ident, 0.0)


def _block_body(x_ref, r1_ref, b1_ref, r2_ref, b2_ref, o_ref, *, sh, w, h):
    c = _LANE
    hw = h * w
    hwp = ((hw + c - 1) // c) * c

    xf = x_ref[...]
    xfp = jnp.concatenate([xf, jnp.zeros((c, hwp - hw), jnp.float32)], axis=1)
    xt = jnp.transpose(xfp)
    xs_all = xt[0:hw].astype(jnp.bfloat16).reshape(h, w, c)
    zr = jnp.zeros((2, w, c), jnp.bfloat16)
    xs_pad = jnp.concatenate([zr, xs_all, zr], axis=0)

    ws = (r1_ref[...], b1_ref[0], r2_ref[...], b2_ref[0])
    results = []
    for t in range(h // sh):
        xs = xs_pad[t * sh:t * sh + sh + 4]
        ident = xt[t * sh * w:(t + 1) * sh * w]
        results.append(_subtile(xs, ident, *ws, t * sh, sh, w, h))
    for t, r in enumerate(results):
        o_ref[t * sh:(t + 1) * sh] = r.reshape(sh, w, c)


def _basic_block(x_nchw, w1, g1, be1, m1, v1, w2, g2, be2, m2, v2, eps=1e-5):
    B, C, H, W = x_nchw.shape
    assert C == _LANE and w1.shape[0] == C
    sh = _SH if H % _SH == 0 else H

    s1 = g1 / jnp.sqrt(v1 + eps)
    s2 = g2 / jnp.sqrt(v2 + eps)
    b1 = (be1 - m1 * s1).reshape(1, C).astype(jnp.float32)
    b2 = (be2 - m2 * s2).reshape(1, C).astype(jnp.float32)

    def prep_w(wt, s):
        whwio = jnp.transpose(wt, (2, 3, 1, 0)) * s[None, None, None, :]
        per_ky = whwio.reshape(3, 3 * C, C).astype(jnp.bfloat16)
        return jnp.concatenate([per_ky[0], per_ky[1], per_ky[2]], axis=1)

    w1f = prep_w(w1, s1)
    w2f = prep_w(w2, s2)

    xflat = jnp.reshape(x_nchw, (B, C, H * W))

    def body(*refs):
        return _block_body(*refs, sh=sh, w=W, h=H)

    out_nhwc = pl.pallas_call(
        body,
        out_shape=jax.ShapeDtypeStruct((B, H, W, C), jnp.float32),
        grid_spec=pltpu.PrefetchScalarGridSpec(
            num_scalar_prefetch=0,
            grid=(B,),
            in_specs=[
                pl.BlockSpec((None, C, H * W), lambda b: (b, 0, 0)),
                pl.BlockSpec((3 * C, 3 * C), lambda b: (0, 0)),
                pl.BlockSpec((1, C), lambda b: (0, 0)),
                pl.BlockSpec((3 * C, 3 * C), lambda b: (0, 0)),
                pl.BlockSpec((1, C), lambda b: (0, 0)),
            ],
            out_specs=pl.BlockSpec((None, H, W, C), lambda b: (b, 0, 0, 0)),
        ),
        compiler_params=pltpu.CompilerParams(
            dimension_semantics=("parallel",)),
    )(xflat, w1f, b1, w2f, b2)

    return jnp.transpose(out_nhwc, (0, 3, 1, 2))


def kernel(x, w1, g1, be1, m1, v1, w2, g2, be2, m2, v2):
    return _basic_block(x, w1, g1, be1, m1, v1, w2, g2, be2, m2, v2)
